# Initial kernel scaffold; baseline (speedup 1.0000x reference)
import jax
import jax.numpy as jnp
from jax.experimental import pallas as pl


def kernel(s, v, edges_ij, r_ij, r_ij_vec, src, dst, W1_s, b1_s, W1_v, W2_s, b2_s, W2_v, W3_s, b3_s, W3_v, Wself_s, Wself_v, R1_w1, R1_b1, R1_w2, R1_b2, R2_w1, R2_b1, R2_w2, R2_b2):
    raise NotImplementedError("write your pallas kernel here")


# speedup vs baseline: 10.4097x; 10.4097x over previous
import functools

import jax
import jax.numpy as jnp
from jax import lax
from jax.experimental import pallas as pl
from jax.experimental.pallas import tpu as pltpu
from jax.experimental.pallas import tpu_sc as plsc

_NC = 128
_D = 4 * _NC
_CH = 128
_SC_CORES = 2
_SC_TILES = 16
_B_EDGE = 1280
_B_NODE = 2000


def _sc_gather_probe(table, idx):
    rows_total = idx.shape[0]
    d = table.shape[1]
    mesh = plsc.VectorSubcoreMesh(core_axis_name="c", subcore_axis_name="s")
    n_iter = rows_total // (_CH * _SC_CORES * _SC_TILES)

    @functools.partial(
        pl.kernel,
        out_type=jax.ShapeDtypeStruct((rows_total, d), jnp.float32),
        mesh=mesh,
        scratch_types=[
            pltpu.VMEM((_CH,), jnp.int32),
            pltpu.VMEM((_CH, d), jnp.float32),
            pltpu.SemaphoreType.DMA,
        ],
    )
    def k(table_hbm, idx_hbm, out_hbm, idx_v, rows_v, sem):
        wid = lax.axis_index("s") * _SC_CORES + lax.axis_index("c")

        @pl.loop(0, n_iter)
        def _(j):
            base = (j * _SC_CORES * _SC_TILES + wid) * _CH
            pltpu.sync_copy(idx_hbm.at[pl.ds(base, _CH)], idx_v)
            pltpu.async_copy(table_hbm.at[idx_v], rows_v, sem).wait()
            pltpu.sync_copy(rows_v, out_hbm.at[pl.ds(base, _CH)])

    return k(table, idx)


def _sc_gather(table, idx):
    rows_total = idx.shape[0]
    d = table.shape[1]
    n_chunks = rows_total // _CH
    nw = _SC_CORES * _SC_TILES
    per = n_chunks // nw
    assert per * nw == n_chunks and n_chunks * _CH == rows_total
    mesh = plsc.VectorSubcoreMesh(core_axis_name="c", subcore_axis_name="s")

    @functools.partial(
        pl.kernel,
        out_type=jax.ShapeDtypeStruct((rows_total, d), jnp.float32),
        mesh=mesh,
        scratch_types=[
            pltpu.VMEM((_CH,), jnp.int32),
            pltpu.VMEM((_CH, d), jnp.float32),
            pltpu.SemaphoreType.DMA,
        ],
    )
    def k(table_hbm, idx_hbm, out_hbm, idx_v, rows_v, sem):
        wid = lax.axis_index("s") * _SC_CORES + lax.axis_index("c")

        def do_chunk(chunk):
            row0 = chunk * _CH
            pltpu.sync_copy(idx_hbm.at[pl.ds(row0, _CH)], idx_v)
            pltpu.async_copy(table_hbm.at[idx_v], rows_v, sem).wait()
            pltpu.sync_copy(rows_v, out_hbm.at[pl.ds(row0, _CH)])

        @pl.loop(0, per)
        def _(j):
            do_chunk(j * nw + wid)

    return k(table, idx)


def _sc_scatter_add(edge_out, dst, zeros_n):
    e_total, d = edge_out.shape
    n_pad = zeros_n.shape[0]
    groups = d // _NC
    n_chunks = e_total // _CH
    nw = _SC_CORES * _SC_TILES
    per = n_chunks // nw
    rows_per_tile = n_pad // _SC_TILES
    assert per * nw == n_chunks and rows_per_tile * _SC_TILES == n_pad
    assert rows_per_tile % 8 == 0
    mesh = plsc.VectorSubcoreMesh(core_axis_name="c", subcore_axis_name="s")

    @functools.partial(
        pl.kernel,
        out_type=jax.ShapeDtypeStruct((_SC_CORES, n_pad, d), jnp.float32),
        mesh=mesh,
        scratch_types=[
            pltpu.VMEM_SHARED((n_pad, _NC), jnp.float32),
            pltpu.VMEM((_CH,), jnp.int32),
            pltpu.VMEM((_CH, _NC), jnp.float32),
        ],
    )
    def k(edge_hbm, dst_hbm, zeros_hbm, out_hbm, acc, idx_v, stage):
        cid = lax.axis_index("c")
        sid = lax.axis_index("s")
        wid = sid * _SC_CORES + cid
        row0 = sid * rows_per_tile

        for g in range(groups):
            pltpu.sync_copy(zeros_hbm.at[pl.ds(row0, rows_per_tile)],
                            acc.at[pl.ds(row0, rows_per_tile)])
            plsc.subcore_barrier()

            @pl.loop(0, per)
            def _(j):
                e0 = (j * nw + wid) * _CH
                pltpu.sync_copy(dst_hbm.at[pl.ds(e0, _CH)], idx_v)
                pltpu.sync_copy(
                    edge_hbm.at[pl.ds(e0, _CH), pl.ds(g * _NC, _NC)], stage)
                pltpu.sync_copy(stage, acc.at[idx_v], add=True)

            plsc.subcore_barrier()
            pltpu.sync_copy(
                acc.at[pl.ds(row0, rows_per_tile)],
                out_hbm.at[cid, pl.ds(row0, rows_per_tile), pl.ds(g * _NC, _NC)])
            plsc.subcore_barrier()

    return k(edge_out, dst, zeros_n)


def _silu(x):
    return x * jax.nn.sigmoid(x)


def _tc_edge_chain(g, r2, eij, rv,
                   R1_w1, R1_b1, R1_w2, R1_b2, R2_w1, R2_b1, R2_w2, R2_b2,
                   W1_s, b1_s, W1_v, W2_s, b2_s, W2_v, valid_blocks):
    e_total = r2.shape[0]
    grid = e_total // _B_EDGE
    nc = _NC

    def body(gd_ref, gs_ref, r_ref, e_ref, rv_ref,
             r1w1, r1b1, r1w2, r1b2, r2w1, r2b1, r2w2, r2b2,
             w1s, w1b, w1v, w2s, w2b, w2v, out_ref):
        gd = gd_ref[...]
        gs = gs_ref[...]
        s1 = gd[:, :nc]
        v1 = (gd[:, nc:2 * nc], gd[:, 2 * nc:3 * nc], gd[:, 3 * nc:])
        s2 = gs[:, :nc]
        v2 = (gs[:, nc:2 * nc], gs[:, 2 * nc:3 * nc], gs[:, 3 * nc:])

        f = jnp.concatenate([r_ref[...], e_ref[...]], axis=1)

        def radial(wa, ba, wb, bb):
            h = _silu(jnp.dot(f, wa[...], preferred_element_type=jnp.float32)
                      + ba[...])
            return jnp.dot(h, wb[...], preferred_element_type=jnp.float32) + bb[...]

        w1 = radial(r1w1, r1b1, r1w2, r1b2)
        w1_se, w1_ve = w1[:, :2 * nc], w1[:, 2 * nc:]

        ss = s1 * s2
        vv = v1[0] * v2[0] + v1[1] * v2[1] + v1[2] * v2[2]
        se = jnp.concatenate([ss, vv], axis=1) * w1_se
        se = jnp.dot(se, w1s[...], preferred_element_type=jnp.float32) + w1b[...]
        ve = []
        for c in range(3):
            t = jnp.concatenate([s1 * v2[c], v1[c] * s2], axis=1) * w1_ve
            ve.append(jnp.dot(t, w1v[...], preferred_element_type=jnp.float32))
        a = _silu(se[:, :nc])
        gate = jax.nn.sigmoid(se[:, nc:])
        ve = [gate * x for x in ve]

        rvb = rv_ref[...]
        rc = (rvb[:, 0:1], rvb[:, 1:2], rvb[:, 2:3])
        w2 = radial(r2w1, r2b1, r2w2, r2b2)
        w2_se, w2_ve = w2[:, :2 * nc], w2[:, 2 * nc:]

        vv2 = ve[0] * rc[0] + ve[1] * rc[1] + ve[2] * rc[2]
        se2 = jnp.concatenate([a, vv2], axis=1) * w2_se
        se2 = jnp.dot(se2, w2s[...], preferred_element_type=jnp.float32) + w2b[...]
        ve2 = []
        for c in range(3):
            t = jnp.concatenate([a * rc[c], ve[c]], axis=1) * w2_ve
            ve2.append(jnp.dot(t, w2v[...], preferred_element_type=jnp.float32))
        a2 = _silu(se2[:, :nc])
        g2 = jax.nn.sigmoid(se2[:, nc:])
        out = jnp.concatenate([a2] + [g2 * x for x in ve2], axis=1)
        keep = (pl.program_id(0) < valid_blocks).astype(jnp.float32)
        out_ref[...] = out * keep

    full = lambda shape: pl.BlockSpec(shape, lambda i: (0,) * len(shape))
    return pl.pallas_call(
        body,
        grid=(grid,),
        in_specs=[
            pl.BlockSpec((_B_EDGE, _D), lambda i: (i, 0)),
            pl.BlockSpec((_B_EDGE, _D), lambda i: (i + grid, 0)),
            pl.BlockSpec((_B_EDGE, 1), lambda i: (i, 0)),
            pl.BlockSpec((_B_EDGE, 16), lambda i: (i, 0)),
            pl.BlockSpec((_B_EDGE, 3), lambda i: (i, 0)),
            full((17, 64)), full((1, 64)), full((64, 4 * nc)), full((1, 4 * nc)),
            full((17, 64)), full((1, 64)), full((64, 4 * nc)), full((1, 4 * nc)),
            full((2 * nc, 2 * nc)), full((1, 2 * nc)), full((2 * nc, nc)),
            full((2 * nc, 2 * nc)), full((1, 2 * nc)), full((2 * nc, nc)),
        ],
        out_specs=pl.BlockSpec((_B_EDGE, _D), lambda i: (i, 0)),
        out_shape=jax.ShapeDtypeStruct((e_total, _D), jnp.float32),
        compiler_params=pltpu.CompilerParams(
            dimension_semantics=("arbitrary",)),
    )(g, g, r2, eij, rv,
      R1_w1, R1_b1, R1_w2, R1_b2, R2_w1, R2_b1, R2_w2, R2_b2,
      W1_s, b1_s, W1_v, W2_s, b2_s, W2_v)


def _tc_finale(parts, table, W3_s, b3_s, W3_v, Wself_s, Wself_v, div):
    n_nodes = table.shape[0]
    grid = n_nodes // _B_NODE
    nc = _NC
    inv = 1.0 / div

    def body(p0_ref, p1_ref, t_ref, w3s, w3b, w3v, wss, wsv, os_ref, ov_ref):
        acc = (p0_ref[0] + p1_ref[0]) * inv
        t = t_ref[...]
        os_ref[...] = (
            jnp.dot(acc[:, :nc], w3s[...], preferred_element_type=jnp.float32)
            + w3b[...]
            + jnp.dot(t[:, :nc], wss[...], preferred_element_type=jnp.float32))
        outs = []
        for c in range(3):
            sl = slice((1 + c) * nc, (2 + c) * nc)
            outs.append(
                jnp.dot(acc[:, sl], w3v[...], preferred_element_type=jnp.float32)
                + jnp.dot(t[:, sl], wsv[...], preferred_element_type=jnp.float32))
        ov_ref[...] = jnp.concatenate(outs, axis=1)

    full = lambda shape: pl.BlockSpec(shape, lambda i: (0,) * len(shape))
    return pl.pallas_call(
        body,
        grid=(grid,),
        in_specs=[
            pl.BlockSpec((1, _B_NODE, _D), lambda i: (0, i, 0)),
            pl.BlockSpec((1, _B_NODE, _D), lambda i: (1, i, 0), ),
            pl.BlockSpec((_B_NODE, _D), lambda i: (i, 0)),
            full((nc, nc)), full((1, nc)), full((nc, nc)),
            full((nc, nc)), full((nc, nc)),
        ],
        out_specs=[
            pl.BlockSpec((_B_NODE, nc), lambda i: (i, 0)),
            pl.BlockSpec((_B_NODE, 3 * nc), lambda i: (i, 0)),
        ],
        out_shape=[
            jax.ShapeDtypeStruct((n_nodes, nc), jnp.float32),
            jax.ShapeDtypeStruct((n_nodes, 3 * nc), jnp.float32),
        ],
        compiler_params=pltpu.CompilerParams(
            dimension_semantics=("arbitrary",)),
    )(parts, parts, table, W3_s, b3_s, W3_v, Wself_s, Wself_v)


_BISECT_JNP_GATHER = False
_BISECT_JNP_SCATTER = False


def kernel(s, v, edges_ij, r_ij, r_ij_vec, src, dst,
           W1_s, b1_s, W1_v, W2_s, b2_s, W2_v, W3_s, b3_s, W3_v,
           Wself_s, Wself_v,
           R1_w1, R1_b1, R1_w2, R1_b2, R2_w1, R2_b1, R2_w2, R2_b2):
    n_nodes, nc = s.shape
    e_total = src.shape[0]
    e_pad = -(-e_total // (_B_EDGE * 4)) * (_B_EDGE * 4)
    npad = e_pad - e_total
    valid_blocks = e_total // _B_EDGE
    assert e_total % _B_EDGE == 0 and e_pad % 4096 == 0

    v_t = jnp.transpose(v, (0, 2, 1)).reshape(n_nodes, 3 * nc)
    table = jnp.concatenate([s, v_t], axis=1)
    zpad = jnp.zeros((npad,), jnp.int32)
    dst_p = jnp.concatenate([dst.astype(jnp.int32), zpad])
    idx = jnp.concatenate([dst_p, src.astype(jnp.int32), zpad])

    if _BISECT_JNP_GATHER:
        gathered = table[idx]
    else:
        gathered = _sc_gather(table, idx)

    edge_out = _tc_edge_chain(
        gathered,
        jnp.pad(r_ij[:, None], ((0, npad), (0, 0))),
        jnp.pad(edges_ij, ((0, npad), (0, 0))),
        jnp.pad(r_ij_vec, ((0, npad), (0, 0))),
        R1_w1, R1_b1[None, :], R1_w2, R1_b2[None, :],
        R2_w1, R2_b1[None, :], R2_w2, R2_b2[None, :],
        W1_s, b1_s[None, :], W1_v, W2_s, b2_s[None, :], W2_v,
        valid_blocks)

    n_node_pad = -(-n_nodes // (_SC_TILES * 8)) * (_SC_TILES * 8)
    zeros_n = jnp.zeros((n_node_pad, nc), jnp.float32)
    if _BISECT_JNP_SCATTER:
        p = jnp.zeros((n_node_pad, _D), jnp.float32).at[dst_p].add(edge_out)
        parts = jnp.stack([p, jnp.zeros_like(p)])
    else:
        parts = _sc_scatter_add(edge_out, dst_p, zeros_n)

    s_out, v3 = _tc_finale(parts, table, W3_s, b3_s[None, :], W3_v,
                           Wself_s, Wself_v, 16.0)
    v_out = jnp.transpose(v3.reshape(n_nodes, 3, nc), (0, 2, 1))
    return (s_out, v_out)


# speedup vs baseline: 11.9509x; 1.1481x over previous
import functools

import jax
import jax.numpy as jnp
from jax import lax
from jax.experimental import pallas as pl
from jax.experimental.pallas import tpu as pltpu
from jax.experimental.pallas import tpu_sc as plsc

_NC = 128
_D = 4 * _NC
_CH = 128
_SC_CORES = 2
_SC_TILES = 16
_B_EDGE = 1280
_B_NODE = 2000


def _sc_gather_probe(table, idx):
    rows_total = idx.shape[0]
    d = table.shape[1]
    mesh = plsc.VectorSubcoreMesh(core_axis_name="c", subcore_axis_name="s")
    n_iter = rows_total // (_CH * _SC_CORES * _SC_TILES)

    @functools.partial(
        pl.kernel,
        out_type=jax.ShapeDtypeStruct((rows_total, d), jnp.float32),
        mesh=mesh,
        scratch_types=[
            pltpu.VMEM((_CH,), jnp.int32),
            pltpu.VMEM((_CH, d), jnp.float32),
            pltpu.SemaphoreType.DMA,
        ],
    )
    def k(table_hbm, idx_hbm, out_hbm, idx_v, rows_v, sem):
        wid = lax.axis_index("s") * _SC_CORES + lax.axis_index("c")

        @pl.loop(0, n_iter)
        def _(j):
            base = (j * _SC_CORES * _SC_TILES + wid) * _CH
            pltpu.sync_copy(idx_hbm.at[pl.ds(base, _CH)], idx_v)
            pltpu.async_copy(table_hbm.at[idx_v], rows_v, sem).wait()
            pltpu.sync_copy(rows_v, out_hbm.at[pl.ds(base, _CH)])

    return k(table, idx)


def _sc_gather(table, idx):
    rows_total = idx.shape[0]
    d = table.shape[1]
    dt = table.dtype
    n_chunks = rows_total // _CH
    nw = _SC_CORES * _SC_TILES
    per = n_chunks // nw
    assert per * nw == n_chunks and n_chunks * _CH == rows_total
    mesh = plsc.VectorSubcoreMesh(core_axis_name="c", subcore_axis_name="s")

    @functools.partial(
        pl.kernel,
        out_type=jax.ShapeDtypeStruct((rows_total, d), dt),
        mesh=mesh,
        scratch_types=[
            pltpu.VMEM((_CH,), jnp.int32),
            pltpu.VMEM((_CH, d), dt),
            pltpu.SemaphoreType.DMA,
        ],
    )
    def k(table_hbm, idx_hbm, out_hbm, idx_v, rows_v, sem):
        wid = lax.axis_index("s") * _SC_CORES + lax.axis_index("c")

        def do_chunk(chunk):
            row0 = chunk * _CH
            pltpu.sync_copy(idx_hbm.at[pl.ds(row0, _CH)], idx_v)
            pltpu.async_copy(table_hbm.at[idx_v], rows_v, sem).wait()
            pltpu.sync_copy(rows_v, out_hbm.at[pl.ds(row0, _CH)])

        @pl.loop(0, per)
        def _(j):
            do_chunk(j * nw + wid)

    return k(table, idx)


def _sc_scatter_add(edge_out, dst, zeros_n):
    e_total, d = edge_out.shape
    n_pad = zeros_n.shape[0]
    groups = d // _NC
    n_chunks = e_total // _CH
    nw = _SC_CORES * _SC_TILES
    per = n_chunks // nw
    rows_per_tile = n_pad // _SC_TILES
    assert per * nw == n_chunks and rows_per_tile * _SC_TILES == n_pad
    assert rows_per_tile % 8 == 0
    mesh = plsc.VectorSubcoreMesh(core_axis_name="c", subcore_axis_name="s")

    @functools.partial(
        pl.kernel,
        out_type=jax.ShapeDtypeStruct((_SC_CORES, n_pad, d), jnp.float32),
        mesh=mesh,
        scratch_types=[
            pltpu.VMEM_SHARED((n_pad, _NC), jnp.float32),
            pltpu.VMEM((_CH,), jnp.int32),
            pltpu.VMEM((_CH, _NC), jnp.float32),
        ],
    )
    def k(edge_hbm, dst_hbm, zeros_hbm, out_hbm, acc, idx_v, stage):
        cid = lax.axis_index("c")
        sid = lax.axis_index("s")
        wid = sid * _SC_CORES + cid
        row0 = sid * rows_per_tile

        for g in range(groups):
            pltpu.sync_copy(zeros_hbm.at[pl.ds(row0, rows_per_tile)],
                            acc.at[pl.ds(row0, rows_per_tile)])
            plsc.subcore_barrier()

            @pl.loop(0, per)
            def _(j):
                e0 = (j * nw + wid) * _CH
                pltpu.sync_copy(dst_hbm.at[pl.ds(e0, _CH)], idx_v)
                pltpu.sync_copy(
                    edge_hbm.at[pl.ds(e0, _CH), pl.ds(g * _NC, _NC)], stage)
                pltpu.sync_copy(stage, acc.at[idx_v], add=True)

            plsc.subcore_barrier()
            pltpu.sync_copy(
                acc.at[pl.ds(row0, rows_per_tile)],
                out_hbm.at[cid, pl.ds(row0, rows_per_tile), pl.ds(g * _NC, _NC)])
            plsc.subcore_barrier()

    return k(edge_out, dst, zeros_n)


def _silu(x):
    return x * jax.nn.sigmoid(x)


def _tc_edge_chain(g, r2, eij, rv,
                   R1_w1, R1_b1, R1_w2, R1_b2, R2_w1, R2_b1, R2_w2, R2_b2,
                   W1_s, b1_s, W1_v, W2_s, b2_s, W2_v, valid_blocks):
    e_total = r2.shape[0]
    grid = e_total // _B_EDGE
    nc = _NC

    def body(gd_ref, gs_ref, r_ref, e_ref, rv_ref,
             r1w1, r1b1, r1w2, r1b2, r2w1, r2b1, r2w2, r2b2,
             w1s, w1b, w1v, w2s, w2b, w2v, out_ref):
        def unpack(ref):
            x = ref[...]
            lo = lax.bitcast_convert_type(x << 16, jnp.float32)
            hi = lax.bitcast_convert_type(x & jnp.int32(-65536), jnp.float32)
            return jnp.concatenate([lo, hi], axis=1)

        gd = unpack(gd_ref)
        gs = unpack(gs_ref)
        s1 = gd[:, :nc]
        v1 = (gd[:, nc:2 * nc], gd[:, 2 * nc:3 * nc], gd[:, 3 * nc:])
        s2 = gs[:, :nc]
        v2 = (gs[:, nc:2 * nc], gs[:, 2 * nc:3 * nc], gs[:, 3 * nc:])

        f = jnp.concatenate([r_ref[...], e_ref[...]], axis=1)

        def radial(wa, ba, wb, bb):
            h = _silu(jnp.dot(f, wa[...], preferred_element_type=jnp.float32)
                      + ba[...])
            return jnp.dot(h, wb[...], preferred_element_type=jnp.float32) + bb[...]

        w1 = radial(r1w1, r1b1, r1w2, r1b2)
        w1_se, w1_ve = w1[:, :2 * nc], w1[:, 2 * nc:]

        ss = s1 * s2
        vv = v1[0] * v2[0] + v1[1] * v2[1] + v1[2] * v2[2]
        se = jnp.concatenate([ss, vv], axis=1) * w1_se
        se = jnp.dot(se, w1s[...], preferred_element_type=jnp.float32) + w1b[...]
        ve = []
        for c in range(3):
            t = jnp.concatenate([s1 * v2[c], v1[c] * s2], axis=1) * w1_ve
            ve.append(jnp.dot(t, w1v[...], preferred_element_type=jnp.float32))
        a = _silu(se[:, :nc])
        gate = jax.nn.sigmoid(se[:, nc:])
        ve = [gate * x for x in ve]

        rvb = rv_ref[...]
        rc = (rvb[:, 0:1], rvb[:, 1:2], rvb[:, 2:3])
        w2 = radial(r2w1, r2b1, r2w2, r2b2)
        w2_se, w2_ve = w2[:, :2 * nc], w2[:, 2 * nc:]

        vv2 = ve[0] * rc[0] + ve[1] * rc[1] + ve[2] * rc[2]
        se2 = jnp.concatenate([a, vv2], axis=1) * w2_se
        se2 = jnp.dot(se2, w2s[...], preferred_element_type=jnp.float32) + w2b[...]
        ve2 = []
        for c in range(3):
            t = jnp.concatenate([a * rc[c], ve[c]], axis=1) * w2_ve
            ve2.append(jnp.dot(t, w2v[...], preferred_element_type=jnp.float32))
        a2 = _silu(se2[:, :nc])
        g2 = jax.nn.sigmoid(se2[:, nc:])
        out = jnp.concatenate([a2] + [g2 * x for x in ve2], axis=1)
        keep = (pl.program_id(0) < valid_blocks).astype(jnp.float32)
        out_ref[...] = out * keep

    full = lambda shape: pl.BlockSpec(shape, lambda i: (0,) * len(shape))
    return pl.pallas_call(
        body,
        grid=(grid,),
        in_specs=[
            pl.BlockSpec((_B_EDGE, _D // 2), lambda i: (i, 0)),
            pl.BlockSpec((_B_EDGE, _D // 2), lambda i: (i + grid, 0)),
            pl.BlockSpec((_B_EDGE, 1), lambda i: (i, 0)),
            pl.BlockSpec((_B_EDGE, 16), lambda i: (i, 0)),
            pl.BlockSpec((_B_EDGE, 3), lambda i: (i, 0)),
            full((17, 64)), full((1, 64)), full((64, 4 * nc)), full((1, 4 * nc)),
            full((17, 64)), full((1, 64)), full((64, 4 * nc)), full((1, 4 * nc)),
            full((2 * nc, 2 * nc)), full((1, 2 * nc)), full((2 * nc, nc)),
            full((2 * nc, 2 * nc)), full((1, 2 * nc)), full((2 * nc, nc)),
        ],
        out_specs=pl.BlockSpec((_B_EDGE, _D), lambda i: (i, 0)),
        out_shape=jax.ShapeDtypeStruct((e_total, _D), jnp.float32),
        compiler_params=pltpu.CompilerParams(
            dimension_semantics=("arbitrary",)),
    )(g, g, r2, eij, rv,
      R1_w1, R1_b1, R1_w2, R1_b2, R2_w1, R2_b1, R2_w2, R2_b2,
      W1_s, b1_s, W1_v, W2_s, b2_s, W2_v)


def _tc_finale(parts, table, W3_s, b3_s, W3_v, Wself_s, Wself_v, div):
    n_nodes = table.shape[0]
    grid = n_nodes // _B_NODE
    nc = _NC
    inv = 1.0 / div

    def body(p0_ref, p1_ref, t_ref, w3s, w3b, w3v, wss, wsv, os_ref, ov_ref):
        acc = (p0_ref[0] + p1_ref[0]) * inv
        t = t_ref[...]
        os_ref[...] = (
            jnp.dot(acc[:, :nc], w3s[...], preferred_element_type=jnp.float32)
            + w3b[...]
            + jnp.dot(t[:, :nc], wss[...], preferred_element_type=jnp.float32))
        outs = []
        for c in range(3):
            sl = slice((1 + c) * nc, (2 + c) * nc)
            outs.append(
                jnp.dot(acc[:, sl], w3v[...], preferred_element_type=jnp.float32)
                + jnp.dot(t[:, sl], wsv[...], preferred_element_type=jnp.float32))
        ov_ref[...] = jnp.concatenate(outs, axis=1)

    full = lambda shape: pl.BlockSpec(shape, lambda i: (0,) * len(shape))
    return pl.pallas_call(
        body,
        grid=(grid,),
        in_specs=[
            pl.BlockSpec((1, _B_NODE, _D), lambda i: (0, i, 0)),
            pl.BlockSpec((1, _B_NODE, _D), lambda i: (1, i, 0), ),
            pl.BlockSpec((_B_NODE, _D), lambda i: (i, 0)),
            full((nc, nc)), full((1, nc)), full((nc, nc)),
            full((nc, nc)), full((nc, nc)),
        ],
        out_specs=[
            pl.BlockSpec((_B_NODE, nc), lambda i: (i, 0)),
            pl.BlockSpec((_B_NODE, 3 * nc), lambda i: (i, 0)),
        ],
        out_shape=[
            jax.ShapeDtypeStruct((n_nodes, nc), jnp.float32),
            jax.ShapeDtypeStruct((n_nodes, 3 * nc), jnp.float32),
        ],
        compiler_params=pltpu.CompilerParams(
            dimension_semantics=("arbitrary",)),
    )(parts, parts, table, W3_s, b3_s, W3_v, Wself_s, Wself_v)


_BISECT_JNP_GATHER = False
_BISECT_JNP_SCATTER = False


def kernel(s, v, edges_ij, r_ij, r_ij_vec, src, dst,
           W1_s, b1_s, W1_v, W2_s, b2_s, W2_v, W3_s, b3_s, W3_v,
           Wself_s, Wself_v,
           R1_w1, R1_b1, R1_w2, R1_b2, R2_w1, R2_b1, R2_w2, R2_b2):
    n_nodes, nc = s.shape
    e_total = src.shape[0]
    e_pad = -(-e_total // (_B_EDGE * 4)) * (_B_EDGE * 4)
    npad = e_pad - e_total
    valid_blocks = e_total // _B_EDGE
    assert e_total % _B_EDGE == 0 and e_pad % 4096 == 0

    v_t = jnp.transpose(v, (0, 2, 1)).reshape(n_nodes, 3 * nc)
    table = jnp.concatenate([s, v_t], axis=1)
    zpad = jnp.zeros((npad,), jnp.int32)
    dst_p = jnp.concatenate([dst.astype(jnp.int32), zpad])
    idx = jnp.concatenate([dst_p, src.astype(jnp.int32), zpad])

    tb = table.astype(jnp.bfloat16)
    lo16 = lax.bitcast_convert_type(tb[:, :_D // 2], jnp.uint16).astype(jnp.uint32)
    hi16 = lax.bitcast_convert_type(tb[:, _D // 2:], jnp.uint16).astype(jnp.uint32)
    table_pk = lax.bitcast_convert_type((hi16 << 16) | lo16, jnp.int32)
    if _BISECT_JNP_GATHER:
        gathered = table_pk[idx]
    else:
        gathered = _sc_gather(table_pk, idx)

    edge_out = _tc_edge_chain(
        gathered,
        jnp.pad(r_ij[:, None], ((0, npad), (0, 0))),
        jnp.pad(edges_ij, ((0, npad), (0, 0))),
        jnp.pad(r_ij_vec, ((0, npad), (0, 0))),
        R1_w1, R1_b1[None, :], R1_w2, R1_b2[None, :],
        R2_w1, R2_b1[None, :], R2_w2, R2_b2[None, :],
        W1_s, b1_s[None, :], W1_v, W2_s, b2_s[None, :], W2_v,
        valid_blocks)

    n_node_pad = -(-n_nodes // (_SC_TILES * 8)) * (_SC_TILES * 8)
    zeros_n = jnp.zeros((n_node_pad, nc), jnp.float32)
    if _BISECT_JNP_SCATTER:
        p = jnp.zeros((n_node_pad, _D), jnp.float32).at[dst_p].add(edge_out)
        parts = jnp.stack([p, jnp.zeros_like(p)])
    else:
        parts = _sc_scatter_add(edge_out, dst_p, zeros_n)

    s_out, v3 = _tc_finale(parts, table, W3_s, b3_s[None, :], W3_v,
                           Wself_s, Wself_v, 16.0)
    v_out = jnp.transpose(v3.reshape(n_nodes, 3, nc), (0, 2, 1))
    return (s_out, v_out)


# speedup vs baseline: 12.0580x; 1.0090x over previous
import functools

import jax
import jax.numpy as jnp
from jax import lax
from jax.experimental import pallas as pl
from jax.experimental.pallas import tpu as pltpu
from jax.experimental.pallas import tpu_sc as plsc

_NC = 128
_D = 4 * _NC
_CH = 128
_SC_CORES = 2
_SC_TILES = 16
_B_EDGE = 1280
_B_NODE = 2000


def _sc_gather_probe(table, idx):
    rows_total = idx.shape[0]
    d = table.shape[1]
    mesh = plsc.VectorSubcoreMesh(core_axis_name="c", subcore_axis_name="s")
    n_iter = rows_total // (_CH * _SC_CORES * _SC_TILES)

    @functools.partial(
        pl.kernel,
        out_type=jax.ShapeDtypeStruct((rows_total, d), jnp.float32),
        mesh=mesh,
        scratch_types=[
            pltpu.VMEM((_CH,), jnp.int32),
            pltpu.VMEM((_CH, d), jnp.float32),
            pltpu.SemaphoreType.DMA,
        ],
    )
    def k(table_hbm, idx_hbm, out_hbm, idx_v, rows_v, sem):
        wid = lax.axis_index("s") * _SC_CORES + lax.axis_index("c")

        @pl.loop(0, n_iter)
        def _(j):
            base = (j * _SC_CORES * _SC_TILES + wid) * _CH
            pltpu.sync_copy(idx_hbm.at[pl.ds(base, _CH)], idx_v)
            pltpu.async_copy(table_hbm.at[idx_v], rows_v, sem).wait()
            pltpu.sync_copy(rows_v, out_hbm.at[pl.ds(base, _CH)])

    return k(table, idx)


def _sc_gather(table, idx):
    rows_total = idx.shape[0]
    d = table.shape[1]
    dt = table.dtype
    n_chunks = rows_total // _CH
    nw = _SC_CORES * _SC_TILES
    per = n_chunks // nw
    assert per * nw == n_chunks and n_chunks * _CH == rows_total
    mesh = plsc.VectorSubcoreMesh(core_axis_name="c", subcore_axis_name="s")

    assert per >= 4 and per % 2 == 0

    @functools.partial(
        pl.kernel,
        out_type=jax.ShapeDtypeStruct((rows_total, d), dt),
        mesh=mesh,
        scratch_types=[
            pltpu.VMEM((_CH,), jnp.int32),
            pltpu.VMEM((_CH,), jnp.int32),
            pltpu.VMEM((_CH, d), dt),
            pltpu.VMEM((_CH, d), dt),
            pltpu.SemaphoreType.DMA,
            pltpu.SemaphoreType.DMA,
            pltpu.SemaphoreType.DMA,
            pltpu.SemaphoreType.DMA,
            pltpu.SemaphoreType.DMA,
            pltpu.SemaphoreType.DMA,
        ],
    )
    def k(table_hbm, idx_hbm, out_hbm, idx_a, idx_b, rows_a, rows_b,
          is_a, is_b, gs_a, gs_b, ws_a, ws_b):
        wid = lax.axis_index("s") * _SC_CORES + lax.axis_index("c")
        idx = (idx_a, idx_b)
        rows = (rows_a, rows_b)
        isem = (is_a, is_b)
        gsem = (gs_a, gs_b)
        wsem = (ws_a, ws_b)

        def off(j):
            return (j * nw + wid) * _CH

        def out_at(j):
            return out_hbm.at[pl.ds(off(j), _CH)]

        def idx_at(j):
            return idx_hbm.at[pl.ds(off(j), _CH)]

        def step(j, p, first=False, load_idx=True, issue_gather=True):
            q = 1 - p
            pltpu.make_async_copy(table_hbm.at[idx[p]], rows[p],
                                  gsem[p]).wait()
            pltpu.async_copy(rows[p], out_at(j), wsem[p])
            if issue_gather:
                pltpu.make_async_copy(idx_at(j + 1), idx[q], isem[q]).wait()
                if not first:
                    pltpu.make_async_copy(rows[q], out_at(j - 1),
                                          wsem[q]).wait()
                pltpu.async_copy(table_hbm.at[idx[q]], rows[q], gsem[q])
                if load_idx:
                    pltpu.async_copy(idx_at(j + 2), idx[p], isem[p])

        pltpu.sync_copy(idx_at(0), idx_a)
        pltpu.async_copy(table_hbm.at[idx_a], rows_a, gsem[0])
        pltpu.async_copy(idx_at(1), idx_b, isem[1])
        step(0, 0, first=True)

        @pl.loop(0, (per - 4) // 2)
        def _(kk):
            j = 2 * kk + 1
            step(j, 1)
            step(j + 1, 0)

        step(per - 3, 1)
        step(per - 2, 0, load_idx=False)
        step(per - 1, 1, issue_gather=False)
        pltpu.make_async_copy(rows_a, out_at(per - 2), wsem[0]).wait()
        pltpu.make_async_copy(rows_b, out_at(per - 1), wsem[1]).wait()

    return k(table, idx)


def _sc_scatter_add(edge_out, dst, zeros_n):
    e_total, d = edge_out.shape
    n_pad = zeros_n.shape[0]
    groups = d // _NC
    n_chunks = e_total // _CH
    nw = _SC_CORES * _SC_TILES
    per = n_chunks // nw
    rows_per_tile = n_pad // _SC_TILES
    assert per * nw == n_chunks and rows_per_tile * _SC_TILES == n_pad
    assert rows_per_tile % 8 == 0
    mesh = plsc.VectorSubcoreMesh(core_axis_name="c", subcore_axis_name="s")

    @functools.partial(
        pl.kernel,
        out_type=jax.ShapeDtypeStruct((_SC_CORES, n_pad, d), jnp.float32),
        mesh=mesh,
        scratch_types=[
            pltpu.VMEM_SHARED((n_pad, _NC), jnp.float32),
            pltpu.VMEM((_CH,), jnp.int32),
            pltpu.VMEM((_CH, _NC), jnp.float32),
        ],
    )
    def k(edge_hbm, dst_hbm, zeros_hbm, out_hbm, acc, idx_v, stage):
        cid = lax.axis_index("c")
        sid = lax.axis_index("s")
        wid = sid * _SC_CORES + cid
        row0 = sid * rows_per_tile

        for g in range(groups):
            pltpu.sync_copy(zeros_hbm.at[pl.ds(row0, rows_per_tile)],
                            acc.at[pl.ds(row0, rows_per_tile)])
            plsc.subcore_barrier()

            @pl.loop(0, per)
            def _(j):
                e0 = (j * nw + wid) * _CH
                pltpu.sync_copy(dst_hbm.at[pl.ds(e0, _CH)], idx_v)
                pltpu.sync_copy(
                    edge_hbm.at[pl.ds(e0, _CH), pl.ds(g * _NC, _NC)], stage)
                pltpu.sync_copy(stage, acc.at[idx_v], add=True)

            plsc.subcore_barrier()
            pltpu.sync_copy(
                acc.at[pl.ds(row0, rows_per_tile)],
                out_hbm.at[cid, pl.ds(row0, rows_per_tile), pl.ds(g * _NC, _NC)])
            plsc.subcore_barrier()

    return k(edge_out, dst, zeros_n)


def _silu(x):
    return x * jax.nn.sigmoid(x)


def _tc_edge_chain(g, r2, eij, rv,
                   R1_w1, R1_b1, R1_w2, R1_b2, R2_w1, R2_b1, R2_w2, R2_b2,
                   W1_s, b1_s, W1_v, W2_s, b2_s, W2_v, valid_blocks):
    e_total = r2.shape[0]
    grid = e_total // _B_EDGE
    nc = _NC

    def body(gd_ref, gs_ref, r_ref, e_ref, rv_ref,
             r1w1, r1b1, r1w2, r1b2, r2w1, r2b1, r2w2, r2b2,
             w1s, w1b, w1v, w2s, w2b, w2v, out_ref):
        def unpack(ref):
            x = ref[...]
            lo = lax.bitcast_convert_type(x << 16, jnp.float32)
            hi = lax.bitcast_convert_type(x & jnp.int32(-65536), jnp.float32)
            return jnp.concatenate([lo, hi], axis=1)

        gd = unpack(gd_ref)
        gs = unpack(gs_ref)
        s1 = gd[:, :nc]
        v1 = (gd[:, nc:2 * nc], gd[:, 2 * nc:3 * nc], gd[:, 3 * nc:])
        s2 = gs[:, :nc]
        v2 = (gs[:, nc:2 * nc], gs[:, 2 * nc:3 * nc], gs[:, 3 * nc:])

        f = jnp.concatenate([r_ref[...], e_ref[...]], axis=1)

        def radial(wa, ba, wb, bb):
            h = _silu(jnp.dot(f, wa[...], preferred_element_type=jnp.float32)
                      + ba[...])
            return jnp.dot(h, wb[...], preferred_element_type=jnp.float32) + bb[...]

        w1 = radial(r1w1, r1b1, r1w2, r1b2)
        w1_se, w1_ve = w1[:, :2 * nc], w1[:, 2 * nc:]

        ss = s1 * s2
        vv = v1[0] * v2[0] + v1[1] * v2[1] + v1[2] * v2[2]
        se = jnp.concatenate([ss, vv], axis=1) * w1_se
        se = jnp.dot(se, w1s[...], preferred_element_type=jnp.float32) + w1b[...]
        ve = []
        for c in range(3):
            t = jnp.concatenate([s1 * v2[c], v1[c] * s2], axis=1) * w1_ve
            ve.append(jnp.dot(t, w1v[...], preferred_element_type=jnp.float32))
        a = _silu(se[:, :nc])
        gate = jax.nn.sigmoid(se[:, nc:])
        ve = [gate * x for x in ve]

        rvb = rv_ref[...]
        rc = (rvb[:, 0:1], rvb[:, 1:2], rvb[:, 2:3])
        w2 = radial(r2w1, r2b1, r2w2, r2b2)
        w2_se, w2_ve = w2[:, :2 * nc], w2[:, 2 * nc:]

        vv2 = ve[0] * rc[0] + ve[1] * rc[1] + ve[2] * rc[2]
        se2 = jnp.concatenate([a, vv2], axis=1) * w2_se
        se2 = jnp.dot(se2, w2s[...], preferred_element_type=jnp.float32) + w2b[...]
        ve2 = []
        for c in range(3):
            t = jnp.concatenate([a * rc[c], ve[c]], axis=1) * w2_ve
            ve2.append(jnp.dot(t, w2v[...], preferred_element_type=jnp.float32))
        a2 = _silu(se2[:, :nc])
        g2 = jax.nn.sigmoid(se2[:, nc:])
        out = jnp.concatenate([a2] + [g2 * x for x in ve2], axis=1)
        keep = (pl.program_id(0) < valid_blocks).astype(jnp.float32)
        out_ref[...] = out * keep

    full = lambda shape: pl.BlockSpec(shape, lambda i: (0,) * len(shape))
    return pl.pallas_call(
        body,
        grid=(grid,),
        in_specs=[
            pl.BlockSpec((_B_EDGE, _D // 2), lambda i: (i, 0)),
            pl.BlockSpec((_B_EDGE, _D // 2), lambda i: (i + grid, 0)),
            pl.BlockSpec((_B_EDGE, 1), lambda i: (i, 0)),
            pl.BlockSpec((_B_EDGE, 16), lambda i: (i, 0)),
            pl.BlockSpec((_B_EDGE, 3), lambda i: (i, 0)),
            full((17, 64)), full((1, 64)), full((64, 4 * nc)), full((1, 4 * nc)),
            full((17, 64)), full((1, 64)), full((64, 4 * nc)), full((1, 4 * nc)),
            full((2 * nc, 2 * nc)), full((1, 2 * nc)), full((2 * nc, nc)),
            full((2 * nc, 2 * nc)), full((1, 2 * nc)), full((2 * nc, nc)),
        ],
        out_specs=pl.BlockSpec((_B_EDGE, _D), lambda i: (i, 0)),
        out_shape=jax.ShapeDtypeStruct((e_total, _D), jnp.float32),
        compiler_params=pltpu.CompilerParams(
            dimension_semantics=("arbitrary",)),
    )(g, g, r2, eij, rv,
      R1_w1, R1_b1, R1_w2, R1_b2, R2_w1, R2_b1, R2_w2, R2_b2,
      W1_s, b1_s, W1_v, W2_s, b2_s, W2_v)


def _tc_finale(parts, table, W3_s, b3_s, W3_v, Wself_s, Wself_v, div):
    n_nodes = table.shape[0]
    grid = n_nodes // _B_NODE
    nc = _NC
    inv = 1.0 / div

    def body(p0_ref, p1_ref, t_ref, w3s, w3b, w3v, wss, wsv, os_ref, ov_ref):
        acc = (p0_ref[0] + p1_ref[0]) * inv
        t = t_ref[...]
        os_ref[...] = (
            jnp.dot(acc[:, :nc], w3s[...], preferred_element_type=jnp.float32)
            + w3b[...]
            + jnp.dot(t[:, :nc], wss[...], preferred_element_type=jnp.float32))
        outs = []
        for c in range(3):
            sl = slice((1 + c) * nc, (2 + c) * nc)
            outs.append(
                jnp.dot(acc[:, sl], w3v[...], preferred_element_type=jnp.float32)
                + jnp.dot(t[:, sl], wsv[...], preferred_element_type=jnp.float32))
        ov_ref[...] = jnp.concatenate(outs, axis=1)

    full = lambda shape: pl.BlockSpec(shape, lambda i: (0,) * len(shape))
    return pl.pallas_call(
        body,
        grid=(grid,),
        in_specs=[
            pl.BlockSpec((1, _B_NODE, _D), lambda i: (0, i, 0)),
            pl.BlockSpec((1, _B_NODE, _D), lambda i: (1, i, 0), ),
            pl.BlockSpec((_B_NODE, _D), lambda i: (i, 0)),
            full((nc, nc)), full((1, nc)), full((nc, nc)),
            full((nc, nc)), full((nc, nc)),
        ],
        out_specs=[
            pl.BlockSpec((_B_NODE, nc), lambda i: (i, 0)),
            pl.BlockSpec((_B_NODE, 3 * nc), lambda i: (i, 0)),
        ],
        out_shape=[
            jax.ShapeDtypeStruct((n_nodes, nc), jnp.float32),
            jax.ShapeDtypeStruct((n_nodes, 3 * nc), jnp.float32),
        ],
        compiler_params=pltpu.CompilerParams(
            dimension_semantics=("arbitrary",)),
    )(parts, parts, table, W3_s, b3_s, W3_v, Wself_s, Wself_v)


_BISECT_JNP_GATHER = False
_BISECT_JNP_SCATTER = False


def kernel(s, v, edges_ij, r_ij, r_ij_vec, src, dst,
           W1_s, b1_s, W1_v, W2_s, b2_s, W2_v, W3_s, b3_s, W3_v,
           Wself_s, Wself_v,
           R1_w1, R1_b1, R1_w2, R1_b2, R2_w1, R2_b1, R2_w2, R2_b2):
    n_nodes, nc = s.shape
    e_total = src.shape[0]
    e_pad = -(-e_total // (_B_EDGE * 4)) * (_B_EDGE * 4)
    npad = e_pad - e_total
    valid_blocks = e_total // _B_EDGE
    assert e_total % _B_EDGE == 0 and e_pad % 4096 == 0

    v_t = jnp.transpose(v, (0, 2, 1)).reshape(n_nodes, 3 * nc)
    table = jnp.concatenate([s, v_t], axis=1)
    zpad = jnp.zeros((npad,), jnp.int32)
    dst_p = jnp.concatenate([dst.astype(jnp.int32), zpad])
    idx = jnp.concatenate([dst_p, src.astype(jnp.int32), zpad])

    tb = table.astype(jnp.bfloat16)
    lo16 = lax.bitcast_convert_type(tb[:, :_D // 2], jnp.uint16).astype(jnp.uint32)
    hi16 = lax.bitcast_convert_type(tb[:, _D // 2:], jnp.uint16).astype(jnp.uint32)
    table_pk = lax.bitcast_convert_type((hi16 << 16) | lo16, jnp.int32)
    if _BISECT_JNP_GATHER:
        gathered = table_pk[idx]
    else:
        gathered = _sc_gather(table_pk, idx)

    edge_out = _tc_edge_chain(
        gathered,
        jnp.pad(r_ij[:, None], ((0, npad), (0, 0))),
        jnp.pad(edges_ij, ((0, npad), (0, 0))),
        jnp.pad(r_ij_vec, ((0, npad), (0, 0))),
        R1_w1, R1_b1[None, :], R1_w2, R1_b2[None, :],
        R2_w1, R2_b1[None, :], R2_w2, R2_b2[None, :],
        W1_s, b1_s[None, :], W1_v, W2_s, b2_s[None, :], W2_v,
        valid_blocks)

    n_node_pad = -(-n_nodes // (_SC_TILES * 8)) * (_SC_TILES * 8)
    zeros_n = jnp.zeros((n_node_pad, nc), jnp.float32)
    if _BISECT_JNP_SCATTER:
        p = jnp.zeros((n_node_pad, _D), jnp.float32).at[dst_p].add(edge_out)
        parts = jnp.stack([p, jnp.zeros_like(p)])
    else:
        parts = _sc_scatter_add(edge_out, dst_p, zeros_n)

    s_out, v3 = _tc_finale(parts, table, W3_s, b3_s[None, :], W3_v,
                           Wself_s, Wself_v, 16.0)
    v_out = jnp.transpose(v3.reshape(n_nodes, 3, nc), (0, 2, 1))
    return (s_out, v_out)
